# Initial kernel scaffold; baseline (speedup 1.0000x reference)
import jax
import jax.numpy as jnp
from jax.experimental import pallas as pl


def kernel(cat0, cat1, cat2, cat3, cat4, cat5, cat6, cat7, num_features, time, lengths, table0, table1, table2, table3, table4, table5, table6, table7, W, b):
    raise NotImplementedError("write your pallas kernel here")


# speedup vs baseline: 3.7727x; 3.7727x over previous
import functools

import jax
import jax.numpy as jnp
from jax import lax
from jax.experimental import pallas as pl
from jax.experimental.pallas import tpu as pltpu
from jax.experimental.pallas import tpu_sc as plsc

L = 200
B = 1024
V = 100000
NCAT = 8
D = 32
NNUM = 4
NED = 32
ROWS = L * B
NG = NNUM + 1
CY = NG * NED
CTOT = NCAT * D + CY

_info = plsc.get_sparse_core_info()
NC, NS = _info.num_cores, _info.num_subcores
NW = NC * NS
RPW = ROWS // NW
CHUNK = 128
NCHUNK = RPW // CHUNK


def _sc_body(cat0, cat1, cat2, cat3, cat4, cat5, cat6, cat7,
             xpad, wflat, bflat,
             t0, t1, t2, t3, t4, t5, t6, t7,
             out,
             idx0, idx1, idx2, idx3, idx4, idx5, idx6, idx7,
             g0, g1, g2, g3, g4, g5, g6, g7,
             xbuf, ybuf, wbuf, bbuf, gsem):
    cats = [cat0, cat1, cat2, cat3, cat4, cat5, cat6, cat7]
    tables = [t0, t1, t2, t3, t4, t5, t6, t7]
    idxs = [idx0, idx1, idx2, idx3, idx4, idx5, idx6, idx7]
    gbufs = [g0, g1, g2, g3, g4, g5, g6, g7]

    wid = lax.axis_index("s") * NC + lax.axis_index("c")

    pltpu.sync_copy(wflat, wbuf)
    pltpu.sync_copy(bflat, bbuf)
    wv = [wbuf[pl.ds(h * 16, 16)] for h in range(2 * NG)]
    bv = [bbuf[pl.ds(h * 16, 16)] for h in range(2 * NG)]

    def chunk_body(j, _):
        base = wid * RPW + j * CHUNK
        for i in range(NCAT):
            pltpu.sync_copy(cats[i].at[pl.ds(base, CHUNK)], idxs[i])
        cps = [pltpu.async_copy(tables[i].at[idxs[i]], gbufs[i], gsem)
               for i in range(NCAT)]
        pltpu.sync_copy(xpad.at[pl.ds(base, CHUNK), :], xbuf)

        def row_body(r, _c):
            xrow = xbuf[r, :]
            for g in range(NG):
                xv = jnp.full((16,), xrow[g], jnp.float32)
                for h in range(2):
                    q = 2 * g + h
                    ybuf[r, pl.ds(q * 16, 16)] = xv * wv[q] + bv[q]
            return _c
        lax.fori_loop(0, CHUNK, row_body, 0)

        for cp in cps:
            cp.wait()
        for i in range(NCAT):
            pltpu.sync_copy(gbufs[i], out.at[pl.ds(base, CHUNK),
                                             pl.ds(i * D, D)])
        pltpu.sync_copy(ybuf, out.at[pl.ds(base, CHUNK),
                                     pl.ds(NCAT * D, CY)])
        return _

    lax.fori_loop(0, NCHUNK, chunk_body, 0)


@jax.jit
def _sc_call(cats, xpad, tables, wflat, bflat):
    mesh = plsc.VectorSubcoreMesh(core_axis_name="c", subcore_axis_name="s")
    scratch = (
        [pltpu.VMEM((CHUNK,), jnp.int32) for _ in range(NCAT)]
        + [pltpu.VMEM((CHUNK, D), jnp.float32) for _ in range(NCAT)]
        + [pltpu.VMEM((CHUNK, 16), jnp.float32),
           pltpu.VMEM((CHUNK, CY), jnp.float32),
           pltpu.VMEM((CY,), jnp.float32),
           pltpu.VMEM((CY,), jnp.float32),
           pltpu.SemaphoreType.DMA]
    )
    fn = pl.kernel(
        _sc_body,
        out_type=jax.ShapeDtypeStruct((ROWS, CTOT), jnp.float32),
        mesh=mesh,
        scratch_types=scratch,
        compiler_params=pltpu.CompilerParams(use_tc_tiling_on_sc=False),
    )
    return fn(*cats, xpad, wflat, bflat, *tables)


def kernel(cat0, cat1, cat2, cat3, cat4, cat5, cat6, cat7,
           num_features, time, lengths,
           table0, table1, table2, table3, table4, table5, table6, table7,
           W, b):
    cats = [c.reshape(ROWS).astype(jnp.int32)
            for c in (cat0, cat1, cat2, cat3, cat4, cat5, cat6, cat7)]
    tables = [table0, table1, table2, table3, table4, table5, table6, table7]
    xpad = jnp.concatenate(
        [num_features.reshape(ROWS, NNUM),
         time.reshape(ROWS, 1).astype(jnp.float32),
         jnp.zeros((ROWS, 16 - NG), jnp.float32)], axis=1)
    out = _sc_call(cats, xpad, tables,
                   W.reshape(CY), b.reshape(CY))
    return out.reshape(L, B, CTOT)


# speedup vs baseline: 4.0310x; 1.0684x over previous
import functools

import jax
import jax.numpy as jnp
from jax import lax
from jax.experimental import pallas as pl
from jax.experimental.pallas import tpu as pltpu
from jax.experimental.pallas import tpu_sc as plsc

L = 200
B = 1024
V = 100000
NCAT = 8
D = 32
NNUM = 4
NED = 32
ROWS = L * B
NG = NNUM + 1
CY = NG * NED
CTOT = NCAT * D + CY

_info = plsc.get_sparse_core_info()
NC, NS = _info.num_cores, _info.num_subcores
NW = NC * NS
RPW = ROWS // NW
CHUNK = 128
NCHUNK = RPW // CHUNK


def _sc_body(cat0, cat1, cat2, cat3, cat4, cat5, cat6, cat7,
             xpad, wflat, bflat,
             t0, t1, t2, t3, t4, t5, t6, t7,
             out,
             ia0, ia1, ia2, ia3, ia4, ia5, ia6, ia7,
             ib0, ib1, ib2, ib3, ib4, ib5, ib6, ib7,
             ga0, ga1, ga2, ga3, ga4, ga5, ga6, ga7,
             gb0, gb1, gb2, gb3, gb4, gb5, gb6, gb7,
             xa, xb, ya, yb, wbuf, bbuf,
             gsema, gsemb, wsema, wsemb):
    cats = [cat0, cat1, cat2, cat3, cat4, cat5, cat6, cat7]
    tables = [t0, t1, t2, t3, t4, t5, t6, t7]
    sets = [
        dict(idx=[ia0, ia1, ia2, ia3, ia4, ia5, ia6, ia7],
             gbuf=[ga0, ga1, ga2, ga3, ga4, ga5, ga6, ga7],
             xbuf=xa, ybuf=ya, gsem=gsema, wsem=wsema),
        dict(idx=[ib0, ib1, ib2, ib3, ib4, ib5, ib6, ib7],
             gbuf=[gb0, gb1, gb2, gb3, gb4, gb5, gb6, gb7],
             xbuf=xb, ybuf=yb, gsem=gsemb, wsem=wsemb),
    ]

    wid = lax.axis_index("s") * NC + lax.axis_index("c")

    pltpu.sync_copy(wflat, wbuf)
    pltpu.sync_copy(bflat, bbuf)
    wv = [wbuf[pl.ds(h * 16, 16)] for h in range(2 * NG)]
    bv = [bbuf[pl.ds(h * 16, 16)] for h in range(2 * NG)]

    def start(j, s):
        base = wid * RPW + j * CHUNK
        for i in range(NCAT):
            pltpu.sync_copy(cats[i].at[pl.ds(base, CHUNK)], s["idx"][i])
        for i in range(NCAT):
            pltpu.async_copy(tables[i].at[s["idx"][i]], s["gbuf"][i],
                             s["gsem"])
        pltpu.sync_copy(xpad.at[pl.ds(base, CHUNK), :], s["xbuf"])

    def out_slices(j):
        base = wid * RPW + j * CHUNK
        l = base // B
        b0 = base - l * B
        slabs = [out.at[l, pl.ds(b0, CHUNK), pl.ds(i * D, D)]
                 for i in range(NCAT)]
        slabs.append(out.at[l, pl.ds(b0, CHUNK), pl.ds(NCAT * D, CY)])
        return slabs

    def finish(j, s):
        ybuf = s["ybuf"]

        def row_body(r, _c):
            xrow = s["xbuf"][r, :]
            for g in range(NG):
                xv = jnp.full((16,), xrow[g], jnp.float32)
                for h in range(2):
                    q = 2 * g + h
                    ybuf[r, pl.ds(q * 16, 16)] = xv * wv[q] + bv[q]
            return _c
        lax.fori_loop(0, CHUNK, row_body, 0)

        for i in range(NCAT):
            pltpu.make_async_copy(tables[i].at[s["idx"][i]], s["gbuf"][i],
                                  s["gsem"]).wait()
        slabs = out_slices(j)
        for i in range(NCAT):
            pltpu.async_copy(s["gbuf"][i], slabs[i], s["wsem"])
        pltpu.async_copy(ybuf, slabs[NCAT], s["wsem"])

    def drain_writes(j, s):
        slabs = out_slices(j)
        for i in range(NCAT):
            pltpu.make_async_copy(s["gbuf"][i], slabs[i], s["wsem"]).wait()
        pltpu.make_async_copy(s["ybuf"], slabs[NCAT], s["wsem"]).wait()

    start(0, sets[0])
    start(1, sets[1])

    def iter_body(k, _c):
        c0 = 2 * k
        finish(c0, sets[0])
        finish(c0 + 1, sets[1])
        drain_writes(c0, sets[0])

        @pl.when(k < NCHUNK // 2 - 1)
        def _():
            start(c0 + 2, sets[0])
        drain_writes(c0 + 1, sets[1])

        @pl.when(k < NCHUNK // 2 - 1)
        def _():
            start(c0 + 3, sets[1])
        return _c

    lax.fori_loop(0, NCHUNK // 2, iter_body, 0)


@jax.jit
def _sc_call(cats, xpad, tables, wflat, bflat):
    mesh = plsc.VectorSubcoreMesh(core_axis_name="c", subcore_axis_name="s")
    scratch = (
        [pltpu.VMEM((CHUNK,), jnp.int32) for _ in range(2 * NCAT)]
        + [pltpu.VMEM((CHUNK, D), jnp.float32) for _ in range(2 * NCAT)]
        + [pltpu.VMEM((CHUNK, 16), jnp.float32) for _ in range(2)]
        + [pltpu.VMEM((CHUNK, CY), jnp.float32) for _ in range(2)]
        + [pltpu.VMEM((CY,), jnp.float32),
           pltpu.VMEM((CY,), jnp.float32)]
        + [pltpu.SemaphoreType.DMA for _ in range(4)]
    )
    fn = pl.kernel(
        _sc_body,
        out_type=jax.ShapeDtypeStruct((L, B, CTOT), jnp.float32),
        mesh=mesh,
        scratch_types=scratch,
        compiler_params=pltpu.CompilerParams(use_tc_tiling_on_sc=False),
    )
    return fn(*cats, xpad, wflat, bflat, *tables)


def kernel(cat0, cat1, cat2, cat3, cat4, cat5, cat6, cat7,
           num_features, time, lengths,
           table0, table1, table2, table3, table4, table5, table6, table7,
           W, b):
    cats = [c.reshape(ROWS).astype(jnp.int32)
            for c in (cat0, cat1, cat2, cat3, cat4, cat5, cat6, cat7)]
    tables = [table0, table1, table2, table3, table4, table5, table6, table7]
    xpad = jnp.concatenate(
        [num_features.reshape(ROWS, NNUM),
         time.reshape(ROWS, 1).astype(jnp.float32),
         jnp.zeros((ROWS, 16 - NG), jnp.float32)], axis=1)
    return _sc_call(cats, xpad, tables, W.reshape(CY), b.reshape(CY))
